# Initial kernel scaffold; baseline (speedup 1.0000x reference)
import jax
import jax.numpy as jnp
from jax.experimental import pallas as pl


def kernel(x, edge_index, W, bconv):
    raise NotImplementedError("write your pallas kernel here")


# speedup vs baseline: 723.6971x; 723.6971x over previous
import functools

import jax
import jax.numpy as jnp
from jax import lax
from jax.experimental import pallas as pl
from jax.experimental.pallas import tpu as pltpu
from jax.experimental.pallas import tpu_sc as plsc

N = 10000
C = 128
K = 32
COUT = 128

NW = 32
NPW = 320
NPAD = NW * NPW
CHUNK = 4
EPC = CHUNK * K
NCHUNKS = NPW // CHUNK
NV = C // 16


def _sc_gather_max_build():
    mesh = plsc.VectorSubcoreMesh(core_axis_name="c", subcore_axis_name="s")

    @functools.partial(
        pl.kernel,
        out_type=jax.ShapeDtypeStruct((NPAD, C), jnp.float32),
        mesh=mesh,
        scratch_types=[
            pltpu.VMEM((NCHUNKS, EPC), jnp.int32),
            pltpu.VMEM((NCHUNKS, EPC), jnp.int32),
            pltpu.VMEM((EPC, C), jnp.float32),
            pltpu.VMEM((EPC, C), jnp.float32),
            pltpu.VMEM((NPW, C), jnp.float32),
            pltpu.SemaphoreType.DMA,
        ],
    )
    def sc_kernel(xt_hbm, isrc_hbm, idst_hbm, m_hbm,
                  isrc_v, idst_v, src_v, dst_v, m_v, sem):
        wid = lax.axis_index("s") * 2 + lax.axis_index("c")
        pltpu.sync_copy(isrc_hbm.at[wid], isrc_v)
        pltpu.sync_copy(idst_hbm.at[wid], idst_v)

        def chunk_body(ci, carry):
            cp_s = pltpu.async_copy(xt_hbm.at[isrc_v.at[ci]], src_v, sem)
            cp_d = pltpu.async_copy(xt_hbm.at[idst_v.at[ci]], dst_v, sem)
            cp_s.wait()
            cp_d.wait()
            for j in range(CHUNK):
                def kbody(k, accs, _j=j):
                    r = _j * K + k
                    return tuple(
                        jnp.maximum(accs[v],
                                    src_v[r, pl.ds(v * 16, 16)]
                                    - dst_v[r, pl.ds(v * 16, 16)])
                        for v in range(NV))
                init = tuple(jnp.full((16,), -jnp.inf, jnp.float32)
                             for _ in range(NV))
                accs = lax.fori_loop(0, K, kbody, init)
                row = ci * CHUNK + j
                for v in range(NV):
                    m_v[row, pl.ds(v * 16, 16)] = accs[v]
            return carry

        lax.fori_loop(0, NCHUNKS, chunk_body, 0)
        pltpu.sync_copy(m_v, m_hbm.at[pl.ds(wid * NPW, NPW)])

    return sc_kernel


_sc_gather_max = _sc_gather_max_build()


def _tc_body(x_ref, m_ref, w1_ref, w2_ref, b_ref, o_ref):
    acc = lax.dot_general(w1_ref[...], x_ref[...],
                          (((1,), (0,)), ((), ())),
                          preferred_element_type=jnp.float32)
    acc = acc + lax.dot_general(w2_ref[...], m_ref[0:N, :],
                                (((1,), (1,)), ((), ())),
                                preferred_element_type=jnp.float32)
    o_ref[...] = jnp.maximum(acc + b_ref[...], 0.0)


def _tc_matmul(x2d, m, w1, w2, b2):
    return pl.pallas_call(
        _tc_body,
        out_shape=jax.ShapeDtypeStruct((COUT, N), jnp.float32),
    )(x2d, m, w1, w2, b2)


def kernel(x, edge_index, W, bconv):
    x2d = x.reshape(C, N)
    xt = x2d.T
    idx = edge_index.reshape(2, N, K).astype(jnp.int32)
    idx = jnp.pad(idx, ((0, 0), (0, NPAD - N), (0, 0)))
    idx = idx.reshape(2, NW, NCHUNKS, EPC)
    m = _sc_gather_max(xt, idx[0], idx[1])
    w1 = W[:, :C]
    w2 = W[:, C:]
    b2 = bconv.reshape(COUT, 1)
    out = _tc_matmul(x2d, m, w1, w2, b2)
    return out.reshape(1, COUT, N, 1)
